# baseline (speedup 1.0000x reference)
import functools
import numpy as np
import jax
import jax.numpy as jnp
from jax import lax
from jax.experimental import pallas as pl
from jax.experimental.pallas import tpu as pltpu
from jax.experimental.pallas import tpu_sc as plsc

_OUT_H = 7
_OUT_W = 7
_SCALE = 0.125
_G = 2
_O = 8
_NBINS = _OUT_H * _OUT_W
_BPC = 4
_NCHUNK = (_NBINS + _BPC - 1) // _BPC
_RPC = 4 * 4 * _BPC
_L = 16

_GDN = lax.GatherDimensionNumbers(
    offset_dims=(), collapsed_slice_dims=(0,), start_index_map=(0,))


def _gath(v, idx):
    return lax.gather(v, idx[:, None], dimension_numbers=_GDN,
                      slice_sizes=(1,),
                      mode=lax.GatherScatterMode.PROMISE_IN_BOUNDS)


def _bcast(v, lane):
    return _gath(v, jnp.full((_L,), lane, dtype=jnp.int32))


def _make_sc_call(NHW, C, R, H, W):
    CO = C * _NBINS
    rois_per_tile = R // 32

    mesh = plsc.VectorSubcoreMesh(core_axis_name="c", subcore_axis_name="s",
                                  num_cores=2, num_subcores=16)

    @functools.partial(
        pl.kernel,
        out_type=jax.ShapeDtypeStruct((R, CO), jnp.float32),
        mesh=mesh,
        scratch_types=[
            pltpu.VMEM((_L,), jnp.float32),
            pltpu.VMEM((2 * _RPC,), jnp.int32),
            pltpu.VMEM((2 * _RPC,), jnp.float32),
            pltpu.VMEM((2 * _RPC, C // 2), jnp.int32),
            pltpu.VMEM((_NCHUNK * _BPC * C,), jnp.float32),
            pltpu.SemaphoreType.DMA,
            pltpu.SemaphoreType.DMA,
        ],
    )
    def sc_call(table, params, out, param_v, idx_v, wbuf, rows_v, pooled,
                sem0, sem1):
        wid = lax.axis_index("s") * 2 + lax.axis_index("c")
        lane = lax.iota(jnp.int32, _L)
        nj = C // _L

        def roi_body(i, carry):
            r = wid * rois_per_tile + i
            pltpu.sync_copy(params.at[r], param_v)
            pv = param_v[...]
            p_cw = _bcast(pv, 0)
            p_ch = _bcast(pv, 1)
            p_rw = _bcast(pv, 2)
            p_rh = _bcast(pv, 3)
            p_cs = _bcast(pv, 4)
            p_sn = _bcast(pv, 5)
            p_lv = _bcast(pv, 6)
            p_rv = _bcast(pv, 7)
            p_ind = _bcast(pv, 8).astype(jnp.int32)
            p_base = _bcast(pv, 9).astype(jnp.int32)
            p_bh = _bcast(pv, 10)
            p_bw = _bcast(pv, 11)
            lane8 = lane & 7
            perm = (lane - lane8) + ((lane8 - p_ind + 8) & 7)
            permp = (lane - lane8) + ((lane8 - p_ind + 9) & 7)

            def fire(ck, slot, sem):
                o = slot * _RPC
                for v in range(_BPC // 4):
                    q = ck * (4 * _BPC) + v * _L + lane
                    b = q >> 2
                    sub = q & 3
                    ph = (b * 9363) >> 16
                    pw = b - ph * _OUT_W
                    iy = sub >> 1
                    ix = sub & 1
                    yy = -0.5 * p_rh + (ph.astype(jnp.float32)
                                        + (iy.astype(jnp.float32) + 0.5) * 0.5) * p_bh
                    xx = -0.5 * p_rw + (pw.astype(jnp.float32)
                                        + (ix.astype(jnp.float32) + 0.5) * 0.5) * p_bw
                    y = yy * p_cs - xx * p_sn + p_ch
                    x = yy * p_sn + xx * p_cs + p_cw
                    vf = jnp.where((y >= -1.0) & (y <= float(H))
                                   & (x >= -1.0) & (x <= float(W)),
                                   0.25, 0.0)
                    yc = jnp.maximum(y, 0.0)
                    xc = jnp.maximum(x, 0.0)
                    yl0 = yc.astype(jnp.int32)
                    xl0 = xc.astype(jnp.int32)
                    cy = yl0 >= H - 1
                    cx = xl0 >= W - 1
                    ylo = jnp.where(cy, H - 1, yl0)
                    yhi = jnp.minimum(yl0 + 1, H - 1)
                    xlo = jnp.where(cx, W - 1, xl0)
                    xhi = jnp.minimum(xl0 + 1, W - 1)
                    yc = jnp.where(cy, float(H - 1), yc)
                    xc = jnp.where(cx, float(W - 1), xc)
                    ly = yc - ylo.astype(jnp.float32)
                    lx = xc - xlo.astype(jnp.float32)
                    hy = 1.0 - ly
                    hx = 1.0 - lx
                    ws = (hy * hx * vf, hy * lx * vf, ly * hx * vf,
                          ly * lx * vf)
                    rowlo = p_base + ylo * W
                    rowhi = p_base + yhi * W
                    rows = (rowlo + xlo, rowlo + xhi, rowhi + xlo,
                            rowhi + xhi)
                    ov = o + v * 4 * _L
                    for c in range(4):
                        idx_v[pl.ds(ov + c * _L, _L)] = rows[c]
                        wbuf[pl.ds(ov + c * _L, _L)] = ws[c]
                pltpu.async_copy(table.at[idx_v.at[pl.ds(o, _RPC)]],
                                 rows_v.at[pl.ds(o, _RPC)], sem)

            def accum(ck, slot, sem):
                o = slot * _RPC
                pltpu.make_async_copy(table.at[pl.ds(0, _RPC)],
                                      rows_v.at[pl.ds(o, _RPC)], sem).wait()
                for k in range(_BPC):
                    v = (4 * k) // _L
                    ov = o + v * 4 * _L
                    ws = [wbuf[pl.ds(ov + c * _L, _L)] for c in range(4)]
                    accs = [jnp.zeros((_L,), jnp.float32)] * nj
                    for s in range(4):
                        l = 4 * k + s - v * _L
                        for c in range(4):
                            wb = _bcast(ws[c], l)
                            for j2 in range(nj // 2):
                                xi = rows_v[ov + c * _L + l,
                                            pl.ds(_L * j2, _L)]
                                va = lax.bitcast_convert_type(
                                    xi << 16, jnp.float32)
                                vb = lax.bitcast_convert_type(
                                    xi & jnp.int32(-65536), jnp.float32)
                                accs[2 * j2] = accs[2 * j2] + wb * va
                                accs[2 * j2 + 1] = accs[2 * j2 + 1] + wb * vb
                    binoff = (ck * _BPC + k) * C
                    for j in range(nj):
                        av = accs[j]
                        pooled[pl.ds(binoff + _L * j, _L)] = (
                            p_rv * _gath(av, perm) + p_lv * _gath(av, permp))

            fire(0, 0, sem0)

            def pair_body(p, carry2):
                ck0 = 2 * p
                ck1 = ck0 + 1

                @pl.when(ck1 < _NCHUNK)
                def _():
                    fire(ck1, 1, sem1)

                accum(ck0, 0, sem0)

                @pl.when(ck1 + 1 < _NCHUNK)
                def _():
                    fire(ck1 + 1, 0, sem0)

                @pl.when(ck1 < _NCHUNK)
                def _():
                    accum(ck1, 1, sem1)

                return carry2

            lax.fori_loop(0, (_NCHUNK + 1) // 2, pair_body, 0)
            pltpu.sync_copy(pooled.at[pl.ds(0, CO)], out.at[r])
            return carry

        lax.fori_loop(0, rois_per_tile, roi_body, 0)

    return sc_call


def kernel(features, rois):
    N, C, H, W = features.shape
    R = rois.shape[0]
    table = jnp.transpose(features, (0, 2, 3, 1)).reshape(N * H * W, C)
    m = np.arange(C)
    chan_perm = (m // 32) * 32 + (m % 32) // 2 + 16 * (m % 2)
    tb = table[:, chan_perm].astype(jnp.bfloat16)
    t16 = jax.lax.bitcast_convert_type(
        tb.reshape(N * H * W, C // 2, 2), jnp.int16)
    table = ((t16[..., 0].astype(jnp.int32) & 0xFFFF)
             | (t16[..., 1].astype(jnp.int32) << 16))
    b = rois[:, 0].astype(jnp.int32)
    cw = rois[:, 1] * _SCALE
    ch = rois[:, 2] * _SCALE
    rw = jnp.maximum(rois[:, 3] * _SCALE, 1.0)
    rh = jnp.maximum(rois[:, 4] * _SCALE, 1.0)
    theta = rois[:, 5]
    ind_f = theta * (_O / (2.0 * np.pi))
    ind = jnp.floor(ind_f)
    l_var = ind_f - ind
    r_var = 1.0 - l_var
    ind_i = jnp.mod(ind.astype(jnp.int32), _O).astype(jnp.float32)
    base = (b * (H * W)).astype(jnp.float32)
    z = jnp.zeros_like(cw)
    params = jnp.stack(
        [cw, ch, rw, rh, jnp.cos(theta), jnp.sin(theta), l_var, r_var,
         ind_i, base, rh / _OUT_H, rw / _OUT_W, z, z, z, z], axis=1)
    out = _make_sc_call(N * H * W, C, R, H, W)(table, params)
    out = out.reshape(R, _NBINS, C).transpose(0, 2, 1)
    return out.reshape(R, C, _OUT_H, _OUT_W)


# speedup vs baseline: 1.0157x; 1.0157x over previous
import functools
import numpy as np
import jax
import jax.numpy as jnp
from jax import lax
from jax.experimental import pallas as pl
from jax.experimental.pallas import tpu as pltpu
from jax.experimental.pallas import tpu_sc as plsc

_OUT_H = 7
_OUT_W = 7
_SCALE = 0.125
_G = 2
_O = 8
_NBINS = _OUT_H * _OUT_W
_BPC = 4
_NCHUNK = (_NBINS + _BPC - 1) // _BPC
_RPC = 4 * 4 * _BPC
_L = 16

_GDN = lax.GatherDimensionNumbers(
    offset_dims=(), collapsed_slice_dims=(0,), start_index_map=(0,))


def _gath(v, idx):
    return lax.gather(v, idx[:, None], dimension_numbers=_GDN,
                      slice_sizes=(1,),
                      mode=lax.GatherScatterMode.PROMISE_IN_BOUNDS)


def _bcast(v, lane):
    return _gath(v, jnp.full((_L,), lane, dtype=jnp.int32))


def _make_sc_call(NHW, C, R, H, W):
    CO = C * _NBINS
    rois_per_tile = R // 32

    mesh = plsc.VectorSubcoreMesh(core_axis_name="c", subcore_axis_name="s",
                                  num_cores=2, num_subcores=16)

    @functools.partial(
        pl.kernel,
        out_type=jax.ShapeDtypeStruct((R, CO), jnp.float32),
        mesh=mesh,
        scratch_types=[
            pltpu.VMEM((_L,), jnp.float32),
            pltpu.VMEM((2 * _RPC,), jnp.int32),
            pltpu.VMEM((2 * _RPC,), jnp.float32),
            pltpu.VMEM((2 * _RPC, C // 2), jnp.int32),
            pltpu.VMEM((_NCHUNK * _BPC * C,), jnp.float32),
            pltpu.SemaphoreType.DMA,
            pltpu.SemaphoreType.DMA,
        ],
    )
    def sc_call(table, params, out, param_v, idx_v, wbuf, rows_v, pooled,
                sem0, sem1):
        wid = lax.axis_index("s") * 2 + lax.axis_index("c")
        lane = lax.iota(jnp.int32, _L)
        nj = C // _L

        def roi_body(i, carry):
            r = wid * rois_per_tile + i
            pltpu.sync_copy(params.at[r], param_v)
            pv = param_v[...]
            p_cw = _bcast(pv, 0)
            p_ch = _bcast(pv, 1)
            p_rw = _bcast(pv, 2)
            p_rh = _bcast(pv, 3)
            p_cs = _bcast(pv, 4)
            p_sn = _bcast(pv, 5)
            p_lv = _bcast(pv, 6)
            p_rv = _bcast(pv, 7)
            p_ind = _bcast(pv, 8).astype(jnp.int32)
            p_base = _bcast(pv, 9).astype(jnp.int32)
            p_bh = _bcast(pv, 10)
            p_bw = _bcast(pv, 11)
            lane8 = lane & 7
            perm = (lane - lane8) + ((lane8 - p_ind + 8) & 7)
            permp = (lane - lane8) + ((lane8 - p_ind + 9) & 7)

            def fire(ck, slot, sem):
                o = slot * _RPC
                for v in range(_BPC // 4):
                    q = ck * (4 * _BPC) + v * _L + lane
                    b = q >> 2
                    sub = q & 3
                    ph = (b * 9363) >> 16
                    pw = b - ph * _OUT_W
                    iy = sub >> 1
                    ix = sub & 1
                    yy = -0.5 * p_rh + (ph.astype(jnp.float32)
                                        + (iy.astype(jnp.float32) + 0.5) * 0.5) * p_bh
                    xx = -0.5 * p_rw + (pw.astype(jnp.float32)
                                        + (ix.astype(jnp.float32) + 0.5) * 0.5) * p_bw
                    y = yy * p_cs - xx * p_sn + p_ch
                    x = yy * p_sn + xx * p_cs + p_cw
                    vf = jnp.where((y >= -1.0) & (y <= float(H))
                                   & (x >= -1.0) & (x <= float(W)),
                                   0.25, 0.0)
                    yc = jnp.maximum(y, 0.0)
                    xc = jnp.maximum(x, 0.0)
                    yl0 = yc.astype(jnp.int32)
                    xl0 = xc.astype(jnp.int32)
                    cy = yl0 >= H - 1
                    cx = xl0 >= W - 1
                    ylo = jnp.where(cy, H - 1, yl0)
                    yhi = jnp.minimum(yl0 + 1, H - 1)
                    xlo = jnp.where(cx, W - 1, xl0)
                    xhi = jnp.minimum(xl0 + 1, W - 1)
                    yc = jnp.where(cy, float(H - 1), yc)
                    xc = jnp.where(cx, float(W - 1), xc)
                    ly = yc - ylo.astype(jnp.float32)
                    lx = xc - xlo.astype(jnp.float32)
                    hy = 1.0 - ly
                    hx = 1.0 - lx
                    ws = (hy * hx * vf, hy * lx * vf, ly * hx * vf,
                          ly * lx * vf)
                    rowlo = p_base + ylo * W
                    rowhi = p_base + yhi * W
                    rows = (rowlo + xlo, rowlo + xhi, rowhi + xlo,
                            rowhi + xhi)
                    ov = o + v * 4 * _L
                    for c in range(4):
                        idx_v[pl.ds(ov + c * _L, _L)] = rows[c]
                        wbuf[pl.ds(ov + c * _L, _L)] = ws[c]
                pltpu.async_copy(table.at[idx_v.at[pl.ds(o, _RPC)]],
                                 rows_v.at[pl.ds(o, _RPC)], sem)

            def accum(ck, slot, sem):
                o = slot * _RPC
                pltpu.make_async_copy(table.at[pl.ds(0, _RPC)],
                                      rows_v.at[pl.ds(o, _RPC)], sem).wait()
                for k in range(_BPC):
                    v = (4 * k) // _L
                    ov = o + v * 4 * _L
                    ws = [wbuf[pl.ds(ov + c * _L, _L)] for c in range(4)]
                    accs = [jnp.zeros((_L,), jnp.float32)] * nj
                    for s in range(4):
                        l = 4 * k + s - v * _L
                        for c in range(4):
                            wb = ws[c][l]
                            for j2 in range(nj // 2):
                                xi = rows_v[ov + c * _L + l,
                                            pl.ds(_L * j2, _L)]
                                va = lax.bitcast_convert_type(
                                    xi << 16, jnp.float32)
                                vb = lax.bitcast_convert_type(
                                    xi & jnp.int32(-65536), jnp.float32)
                                accs[2 * j2] = accs[2 * j2] + wb * va
                                accs[2 * j2 + 1] = accs[2 * j2 + 1] + wb * vb
                    binoff = (ck * _BPC + k) * C
                    for j in range(nj):
                        av = accs[j]
                        pooled[pl.ds(binoff + _L * j, _L)] = (
                            p_rv * _gath(av, perm) + p_lv * _gath(av, permp))

            fire(0, 0, sem0)

            def pair_body(p, carry2):
                ck0 = 2 * p
                ck1 = ck0 + 1

                @pl.when(ck1 < _NCHUNK)
                def _():
                    fire(ck1, 1, sem1)

                accum(ck0, 0, sem0)

                @pl.when(ck1 + 1 < _NCHUNK)
                def _():
                    fire(ck1 + 1, 0, sem0)

                @pl.when(ck1 < _NCHUNK)
                def _():
                    accum(ck1, 1, sem1)

                return carry2

            lax.fori_loop(0, (_NCHUNK + 1) // 2, pair_body, 0)
            pltpu.sync_copy(pooled.at[pl.ds(0, CO)], out.at[r])
            return carry

        lax.fori_loop(0, rois_per_tile, roi_body, 0)

    return sc_call


def kernel(features, rois):
    N, C, H, W = features.shape
    R = rois.shape[0]
    table = jnp.transpose(features, (0, 2, 3, 1)).reshape(N * H * W, C)
    m = np.arange(C)
    chan_perm = (m // 32) * 32 + (m % 32) // 2 + 16 * (m % 2)
    tb = table[:, chan_perm].astype(jnp.bfloat16)
    t16 = jax.lax.bitcast_convert_type(
        tb.reshape(N * H * W, C // 2, 2), jnp.int16)
    table = ((t16[..., 0].astype(jnp.int32) & 0xFFFF)
             | (t16[..., 1].astype(jnp.int32) << 16))
    b = rois[:, 0].astype(jnp.int32)
    cw = rois[:, 1] * _SCALE
    ch = rois[:, 2] * _SCALE
    rw = jnp.maximum(rois[:, 3] * _SCALE, 1.0)
    rh = jnp.maximum(rois[:, 4] * _SCALE, 1.0)
    theta = rois[:, 5]
    ind_f = theta * (_O / (2.0 * np.pi))
    ind = jnp.floor(ind_f)
    l_var = ind_f - ind
    r_var = 1.0 - l_var
    ind_i = jnp.mod(ind.astype(jnp.int32), _O).astype(jnp.float32)
    base = (b * (H * W)).astype(jnp.float32)
    z = jnp.zeros_like(cw)
    params = jnp.stack(
        [cw, ch, rw, rh, jnp.cos(theta), jnp.sin(theta), l_var, r_var,
         ind_i, base, rh / _OUT_H, rw / _OUT_W, z, z, z, z], axis=1)
    out = _make_sc_call(N * H * W, C, R, H, W)(table, params)
    out = out.reshape(R, _NBINS, C).transpose(0, 2, 1)
    return out.reshape(R, C, _OUT_H, _OUT_W)


# speedup vs baseline: 1.1121x; 1.0949x over previous
import functools
import numpy as np
import jax
import jax.numpy as jnp
from jax import lax
from jax.experimental import pallas as pl
from jax.experimental.pallas import tpu as pltpu
from jax.experimental.pallas import tpu_sc as plsc

_OUT_H = 7
_OUT_W = 7
_SCALE = 0.125
_G = 2
_O = 8
_NBINS = _OUT_H * _OUT_W
_BPC = 4
_NCHUNK = (_NBINS + _BPC - 1) // _BPC
_RPC = 4 * 4 * _BPC
_L = 16

_GDN = lax.GatherDimensionNumbers(
    offset_dims=(), collapsed_slice_dims=(0,), start_index_map=(0,))


def _gath(v, idx):
    return lax.gather(v, idx[:, None], dimension_numbers=_GDN,
                      slice_sizes=(1,),
                      mode=lax.GatherScatterMode.PROMISE_IN_BOUNDS)


def _bcast(v, lane):
    return _gath(v, jnp.full((_L,), lane, dtype=jnp.int32))


def _make_sc_call(NHW, C, R, H, W):
    CO = C * _NBINS
    rois_per_tile = R // 32

    mesh = plsc.VectorSubcoreMesh(core_axis_name="c", subcore_axis_name="s",
                                  num_cores=2, num_subcores=16)

    @functools.partial(
        pl.kernel,
        out_type=jax.ShapeDtypeStruct((R, CO), jnp.float32),
        mesh=mesh,
        scratch_types=[
            pltpu.VMEM((_L,), jnp.float32),
            pltpu.VMEM((2 * _RPC,), jnp.int32),
            pltpu.VMEM((2 * _RPC,), jnp.float32),
            pltpu.VMEM((2 * _RPC, C // 2), jnp.int32),
            pltpu.VMEM((_NCHUNK * _BPC * C,), jnp.float32),
            pltpu.SemaphoreType.DMA,
            pltpu.SemaphoreType.DMA,
        ],
    )
    def sc_call(table, params, out, param_v, idx_v, wbuf, rows_v, pooled,
                sem0, sem1):
        wid = lax.axis_index("s") * 2 + lax.axis_index("c")
        lane = lax.iota(jnp.int32, _L)
        nj = C // _L

        def roi_body(i, carry):
            r = wid * rois_per_tile + i
            pltpu.sync_copy(params.at[r], param_v)
            pv = param_v[...]
            p_cw = _bcast(pv, 0)
            p_ch = _bcast(pv, 1)
            p_rw = _bcast(pv, 2)
            p_rh = _bcast(pv, 3)
            p_cs = _bcast(pv, 4)
            p_sn = _bcast(pv, 5)
            p_lv = _bcast(pv, 6)
            p_rv = _bcast(pv, 7)
            p_ind = _bcast(pv, 8).astype(jnp.int32)
            p_base = _bcast(pv, 9).astype(jnp.int32)
            p_bh = _bcast(pv, 10)
            p_bw = _bcast(pv, 11)
            lane8 = lane & 7
            perm = (lane - lane8) + ((lane8 - p_ind + 8) & 7)
            permp = (lane - lane8) + ((lane8 - p_ind + 9) & 7)

            def fire(ck, slot, sem):
                o = slot * _RPC
                for v in range(_BPC // 4):
                    q = ck * (4 * _BPC) + v * _L + lane
                    b = q >> 2
                    sub = q & 3
                    ph = (b * 9363) >> 16
                    pw = b - ph * _OUT_W
                    iy = sub >> 1
                    ix = sub & 1
                    yy = -0.5 * p_rh + (ph.astype(jnp.float32)
                                        + (iy.astype(jnp.float32) + 0.5) * 0.5) * p_bh
                    xx = -0.5 * p_rw + (pw.astype(jnp.float32)
                                        + (ix.astype(jnp.float32) + 0.5) * 0.5) * p_bw
                    y = yy * p_cs - xx * p_sn + p_ch
                    x = yy * p_sn + xx * p_cs + p_cw
                    vf = jnp.where((y >= -1.0) & (y <= float(H))
                                   & (x >= -1.0) & (x <= float(W)),
                                   0.25, 0.0)
                    yc = jnp.maximum(y, 0.0)
                    xc = jnp.maximum(x, 0.0)
                    yl0 = yc.astype(jnp.int32)
                    xl0 = xc.astype(jnp.int32)
                    cy = yl0 >= H - 1
                    cx = xl0 >= W - 1
                    ylo = jnp.where(cy, H - 1, yl0)
                    yhi = jnp.minimum(yl0 + 1, H - 1)
                    xlo = jnp.where(cx, W - 1, xl0)
                    xhi = jnp.minimum(xl0 + 1, W - 1)
                    yc = jnp.where(cy, float(H - 1), yc)
                    xc = jnp.where(cx, float(W - 1), xc)
                    ly = yc - ylo.astype(jnp.float32)
                    lx = xc - xlo.astype(jnp.float32)
                    hy = 1.0 - ly
                    hx = 1.0 - lx
                    ws = (hy * hx * vf, hy * lx * vf, ly * hx * vf,
                          ly * lx * vf)
                    rowlo = p_base + ylo * W
                    rowhi = p_base + yhi * W
                    rows = (rowlo + xlo, rowlo + xhi, rowhi + xlo,
                            rowhi + xhi)
                    ov = o + v * 4 * _L
                    for c in range(4):
                        idx_v[pl.ds(ov + c * _L, _L)] = rows[c]
                        wbuf[pl.ds(ov + c * _L, _L)] = ws[c]
                pltpu.async_copy(table.at[idx_v.at[pl.ds(o, _RPC)]],
                                 rows_v.at[pl.ds(o, _RPC)], sem)

            def accum(ck, slot, sem):
                o = slot * _RPC
                pltpu.make_async_copy(table.at[pl.ds(0, _RPC)],
                                      rows_v.at[pl.ds(o, _RPC)], sem).wait()
                for k in range(_BPC):
                    v = (4 * k) // _L
                    ov = o + v * 4 * _L
                    ws = [wbuf[pl.ds(ov + c * _L, _L)] for c in range(4)]
                    accs = [jnp.zeros((_L,), jnp.float32)] * nj
                    for s in range(4):
                        l = 4 * k + s - v * _L
                        for c in range(4):
                            wb = ws[c][l]
                            for j2 in range(nj // 2):
                                xi = rows_v[ov + c * _L + l,
                                            pl.ds(_L * j2, _L)]
                                va = lax.bitcast_convert_type(
                                    xi << 16, jnp.float32)
                                vb = lax.bitcast_convert_type(xi, jnp.float32)
                                accs[2 * j2] = accs[2 * j2] + wb * va
                                accs[2 * j2 + 1] = accs[2 * j2 + 1] + wb * vb
                    binoff = (ck * _BPC + k) * C
                    for j in range(nj):
                        av = accs[j]
                        pooled[pl.ds(binoff + _L * j, _L)] = (
                            p_rv * _gath(av, perm) + p_lv * _gath(av, permp))

            fire(0, 0, sem0)

            def pair_body(p, carry2):
                ck0 = 2 * p
                ck1 = ck0 + 1

                @pl.when(ck1 < _NCHUNK)
                def _():
                    fire(ck1, 1, sem1)

                accum(ck0, 0, sem0)

                @pl.when(ck1 + 1 < _NCHUNK)
                def _():
                    fire(ck1 + 1, 0, sem0)

                @pl.when(ck1 < _NCHUNK)
                def _():
                    accum(ck1, 1, sem1)

                return carry2

            lax.fori_loop(0, (_NCHUNK + 1) // 2, pair_body, 0)
            pltpu.sync_copy(pooled.at[pl.ds(0, CO)], out.at[r])
            return carry

        lax.fori_loop(0, rois_per_tile, roi_body, 0)

    return sc_call


def kernel(features, rois):
    N, C, H, W = features.shape
    R = rois.shape[0]
    table = jnp.transpose(features, (0, 2, 3, 1)).reshape(N * H * W, C)
    m = np.arange(C)
    chan_perm = (m // 32) * 32 + (m % 32) // 2 + 16 * (m % 2)
    tb = table[:, chan_perm].astype(jnp.bfloat16)
    t16 = jax.lax.bitcast_convert_type(
        tb.reshape(N * H * W, C // 2, 2), jnp.int16)
    table = ((t16[..., 0].astype(jnp.int32) & 0xFFFF)
             | (t16[..., 1].astype(jnp.int32) << 16))
    b = rois[:, 0].astype(jnp.int32)
    cw = rois[:, 1] * _SCALE
    ch = rois[:, 2] * _SCALE
    rw = jnp.maximum(rois[:, 3] * _SCALE, 1.0)
    rh = jnp.maximum(rois[:, 4] * _SCALE, 1.0)
    theta = rois[:, 5]
    ind_f = theta * (_O / (2.0 * np.pi))
    ind = jnp.floor(ind_f)
    l_var = ind_f - ind
    r_var = 1.0 - l_var
    ind_i = jnp.mod(ind.astype(jnp.int32), _O).astype(jnp.float32)
    base = (b * (H * W)).astype(jnp.float32)
    z = jnp.zeros_like(cw)
    params = jnp.stack(
        [cw, ch, rw, rh, jnp.cos(theta), jnp.sin(theta), l_var, r_var,
         ind_i, base, rh / _OUT_H, rw / _OUT_W, z, z, z, z], axis=1)
    out = _make_sc_call(N * H * W, C, R, H, W)(table, params)
    out = out.reshape(R, _NBINS, C).transpose(0, 2, 1)
    return out.reshape(R, C, _OUT_H, _OUT_W)


# speedup vs baseline: 1.1171x; 1.0045x over previous
import functools
import numpy as np
import jax
import jax.numpy as jnp
from jax import lax
from jax.experimental import pallas as pl
from jax.experimental.pallas import tpu as pltpu
from jax.experimental.pallas import tpu_sc as plsc

_OUT_H = 7
_OUT_W = 7
_SCALE = 0.125
_G = 2
_O = 8
_NBINS = _OUT_H * _OUT_W
_BPC = 4
_NCHUNK = (_NBINS + _BPC - 1) // _BPC
_RPC = 4 * 4 * _BPC
_L = 16

_GDN = lax.GatherDimensionNumbers(
    offset_dims=(), collapsed_slice_dims=(0,), start_index_map=(0,))


def _gath(v, idx):
    return lax.gather(v, idx[:, None], dimension_numbers=_GDN,
                      slice_sizes=(1,),
                      mode=lax.GatherScatterMode.PROMISE_IN_BOUNDS)


def _bcast(v, lane):
    return _gath(v, jnp.full((_L,), lane, dtype=jnp.int32))


def _make_sc_call(NHW, C, R, H, W):
    CO = C * _NBINS
    rois_per_tile = R // 32

    mesh = plsc.VectorSubcoreMesh(core_axis_name="c", subcore_axis_name="s",
                                  num_cores=2, num_subcores=16)

    @functools.partial(
        pl.kernel,
        out_type=jax.ShapeDtypeStruct((R, CO), jnp.float32),
        mesh=mesh,
        scratch_types=[
            pltpu.VMEM((_L,), jnp.float32),
            pltpu.VMEM((2 * _RPC,), jnp.int32),
            pltpu.VMEM((2 * _RPC,), jnp.float32),
            pltpu.VMEM((2 * _RPC, C // 2), jnp.int32),
            pltpu.VMEM((2 * _NCHUNK * _BPC * C,), jnp.float32),
            pltpu.SemaphoreType.DMA,
            pltpu.SemaphoreType.DMA,
            pltpu.SemaphoreType.DMA,
        ],
    )
    def sc_call(table, params, out, param_v, idx_v, wbuf, rows_v, pooled,
                sem0, sem1, sem2):
        wid = lax.axis_index("s") * 2 + lax.axis_index("c")
        lane = lax.iota(jnp.int32, _L)
        nj = C // _L

        PSZ = _NCHUNK * _BPC * C

        def roi_body(i, carry):
            r = wid * rois_per_tile + i
            poff = (i & 1) * PSZ

            @pl.when(i >= 2)
            def _():
                pltpu.make_async_copy(pooled.at[pl.ds(poff, CO)],
                                      out.at[r], sem2).wait()

            pltpu.sync_copy(params.at[r], param_v)
            pv = param_v[...]
            p_cw = _bcast(pv, 0)
            p_ch = _bcast(pv, 1)
            p_rw = _bcast(pv, 2)
            p_rh = _bcast(pv, 3)
            p_cs = _bcast(pv, 4)
            p_sn = _bcast(pv, 5)
            p_lv = _bcast(pv, 6)
            p_rv = _bcast(pv, 7)
            p_ind = _bcast(pv, 8).astype(jnp.int32)
            p_base = _bcast(pv, 9).astype(jnp.int32)
            p_bh = _bcast(pv, 10)
            p_bw = _bcast(pv, 11)
            lane8 = lane & 7
            perm = (lane - lane8) + ((lane8 - p_ind + 8) & 7)
            permp = (lane - lane8) + ((lane8 - p_ind + 9) & 7)

            def fire(ck, slot, sem):
                o = slot * _RPC
                for v in range(_BPC // 4):
                    q = ck * (4 * _BPC) + v * _L + lane
                    b = q >> 2
                    sub = q & 3
                    ph = (b * 9363) >> 16
                    pw = b - ph * _OUT_W
                    iy = sub >> 1
                    ix = sub & 1
                    yy = -0.5 * p_rh + (ph.astype(jnp.float32)
                                        + (iy.astype(jnp.float32) + 0.5) * 0.5) * p_bh
                    xx = -0.5 * p_rw + (pw.astype(jnp.float32)
                                        + (ix.astype(jnp.float32) + 0.5) * 0.5) * p_bw
                    y = yy * p_cs - xx * p_sn + p_ch
                    x = yy * p_sn + xx * p_cs + p_cw
                    vf = jnp.where((y >= -1.0) & (y <= float(H))
                                   & (x >= -1.0) & (x <= float(W)),
                                   0.25, 0.0)
                    yc = jnp.maximum(y, 0.0)
                    xc = jnp.maximum(x, 0.0)
                    yl0 = yc.astype(jnp.int32)
                    xl0 = xc.astype(jnp.int32)
                    cy = yl0 >= H - 1
                    cx = xl0 >= W - 1
                    ylo = jnp.where(cy, H - 1, yl0)
                    yhi = jnp.minimum(yl0 + 1, H - 1)
                    xlo = jnp.where(cx, W - 1, xl0)
                    xhi = jnp.minimum(xl0 + 1, W - 1)
                    yc = jnp.where(cy, float(H - 1), yc)
                    xc = jnp.where(cx, float(W - 1), xc)
                    ly = yc - ylo.astype(jnp.float32)
                    lx = xc - xlo.astype(jnp.float32)
                    hy = 1.0 - ly
                    hx = 1.0 - lx
                    ws = (hy * hx * vf, hy * lx * vf, ly * hx * vf,
                          ly * lx * vf)
                    rowlo = p_base + ylo * W
                    rowhi = p_base + yhi * W
                    rows = (rowlo + xlo, rowlo + xhi, rowhi + xlo,
                            rowhi + xhi)
                    ov = o + v * 4 * _L
                    for c in range(4):
                        idx_v[pl.ds(ov + c * _L, _L)] = rows[c]
                        wbuf[pl.ds(ov + c * _L, _L)] = ws[c]
                pltpu.async_copy(table.at[idx_v.at[pl.ds(o, _RPC)]],
                                 rows_v.at[pl.ds(o, _RPC)], sem)

            def accum(ck, slot, sem):
                o = slot * _RPC
                pltpu.make_async_copy(table.at[pl.ds(0, _RPC)],
                                      rows_v.at[pl.ds(o, _RPC)], sem).wait()
                for k in range(_BPC):
                    v = (4 * k) // _L
                    ov = o + v * 4 * _L
                    ws = [wbuf[pl.ds(ov + c * _L, _L)] for c in range(4)]
                    accs = [jnp.zeros((_L,), jnp.float32)] * nj
                    for s in range(4):
                        l = 4 * k + s - v * _L
                        for c in range(4):
                            wb = ws[c][l]
                            for j2 in range(nj // 2):
                                xi = rows_v[ov + c * _L + l,
                                            pl.ds(_L * j2, _L)]
                                va = lax.bitcast_convert_type(
                                    xi << 16, jnp.float32)
                                vb = lax.bitcast_convert_type(xi, jnp.float32)
                                accs[2 * j2] = accs[2 * j2] + wb * va
                                accs[2 * j2 + 1] = accs[2 * j2 + 1] + wb * vb
                    binoff = poff + (ck * _BPC + k) * C
                    for j in range(nj):
                        av = accs[j]
                        pooled[pl.ds(binoff + _L * j, _L)] = (
                            p_rv * _gath(av, perm) + p_lv * _gath(av, permp))

            fire(0, 0, sem0)

            def pair_body(p, carry2):
                ck0 = 2 * p
                ck1 = ck0 + 1

                @pl.when(ck1 < _NCHUNK)
                def _():
                    fire(ck1, 1, sem1)

                accum(ck0, 0, sem0)

                @pl.when(ck1 + 1 < _NCHUNK)
                def _():
                    fire(ck1 + 1, 0, sem0)

                @pl.when(ck1 < _NCHUNK)
                def _():
                    accum(ck1, 1, sem1)

                return carry2

            lax.fori_loop(0, (_NCHUNK + 1) // 2, pair_body, 0)
            pltpu.async_copy(pooled.at[pl.ds(poff, CO)], out.at[r], sem2)
            return carry

        lax.fori_loop(0, rois_per_tile, roi_body, 0)
        for _ in range(2):
            pltpu.make_async_copy(pooled.at[pl.ds(0, CO)],
                                  out.at[wid * rois_per_tile], sem2).wait()

    return sc_call


def kernel(features, rois):
    N, C, H, W = features.shape
    R = rois.shape[0]
    table = jnp.transpose(features, (0, 2, 3, 1)).reshape(N * H * W, C)
    m = np.arange(C)
    chan_perm = (m // 32) * 32 + (m % 32) // 2 + 16 * (m % 2)
    tb = table[:, chan_perm].astype(jnp.bfloat16)
    t16 = jax.lax.bitcast_convert_type(
        tb.reshape(N * H * W, C // 2, 2), jnp.int16)
    table = ((t16[..., 0].astype(jnp.int32) & 0xFFFF)
             | (t16[..., 1].astype(jnp.int32) << 16))
    b = rois[:, 0].astype(jnp.int32)
    cw = rois[:, 1] * _SCALE
    ch = rois[:, 2] * _SCALE
    rw = jnp.maximum(rois[:, 3] * _SCALE, 1.0)
    rh = jnp.maximum(rois[:, 4] * _SCALE, 1.0)
    theta = rois[:, 5]
    ind_f = theta * (_O / (2.0 * np.pi))
    ind = jnp.floor(ind_f)
    l_var = ind_f - ind
    r_var = 1.0 - l_var
    ind_i = jnp.mod(ind.astype(jnp.int32), _O).astype(jnp.float32)
    base = (b * (H * W)).astype(jnp.float32)
    z = jnp.zeros_like(cw)
    params = jnp.stack(
        [cw, ch, rw, rh, jnp.cos(theta), jnp.sin(theta), l_var, r_var,
         ind_i, base, rh / _OUT_H, rw / _OUT_W, z, z, z, z], axis=1)
    out = _make_sc_call(N * H * W, C, R, H, W)(table, params)
    out = out.reshape(R, _NBINS, C).transpose(0, 2, 1)
    return out.reshape(R, C, _OUT_H, _OUT_W)
